# baseline (speedup 1.0000x reference)
import jax
import jax.numpy as jnp
from jax.experimental import pallas as pl
from jax.experimental.pallas import tpu as pltpu

_B = 8
_N = 512
_NUM_NEIGHBORS = 9

_BM = 512
_BK = 2048

_PREC = jax.lax.Precision.HIGHEST


def _phase1_kernel(x_ref, yt_ref, scores_ref, loc_ref, minv_s, mini_s):
    k = pl.program_id(1)
    nk = pl.num_programs(1)
    x = x_ref[...]
    yt = yt_ref[...]
    yn = jnp.sum(yt * yt, axis=0)
    xy = jax.lax.dot_general(
        x, yt, (((1,), (0,)), ((), ())),
        preferred_element_type=jnp.float32, precision=_PREC)
    part = yn[None, :] - 2.0 * xy
    pmin = jnp.min(part, axis=1)
    iota = jax.lax.broadcasted_iota(jnp.int32, part.shape, 1)
    big = jnp.int32(part.shape[1])
    pidx = jnp.min(jnp.where(part <= pmin[:, None], iota, big), axis=1)
    gidx = k * _BK + pidx

    @pl.when(k == 0)
    def _():
        minv_s[...] = pmin[:, None]
        mini_s[...] = gidx[:, None]

    @pl.when(k > 0)
    def _():
        better = pmin[:, None] < minv_s[...]
        minv_s[...] = jnp.where(better, pmin[:, None], minv_s[...])
        mini_s[...] = jnp.where(better, gidx[:, None], mini_s[...])

    @pl.when(k == nk - 1)
    def _():
        xn = jnp.sum(x * x, axis=1, keepdims=True)
        scores_ref[...] = jnp.sqrt(jnp.clip(xn + minv_s[...], 1e-12, None))
        loc_ref[...] = mini_s[...]


def _topk_merge(keys, pays, kk):
    width = keys.shape[1]
    iota = jax.lax.broadcasted_iota(jnp.int32, keys.shape, 1)
    out_k = []
    out_p = []
    for _ in range(kk):
        m = jnp.min(keys, axis=1)
        sel = jnp.min(jnp.where(keys <= m[:, None], iota, width), axis=1)
        onehot = iota == sel[:, None]
        pay = jnp.sum(jnp.where(onehot, pays, 0.0), axis=1)
        out_k.append(m)
        out_p.append(pay)
        keys = jnp.where(onehot, jnp.float32(jnp.inf), keys)
    return jnp.stack(out_k, axis=1), jnp.stack(out_p, axis=1)


def _phase2_kernel(sc_ref, loc_ref, emb_ref, yt_ref, pred_ref,
                   score_s, nnidx_s, mf_s, nnt_s, topk_s, topp_s):
    p = pl.program_id(0)
    k = pl.program_id(1)
    nk = pl.num_programs(1)
    kk = _NUM_NEIGHBORS

    @pl.when(jnp.logical_and(p == 0, k == 0))
    def _():
        s = sc_ref[...]
        iota_n = jax.lax.broadcasted_iota(jnp.int32, s.shape, 1)
        smax = jnp.max(s, axis=1)
        mp = jnp.min(jnp.where(s >= smax[:, None], iota_n, _N), axis=1)
        onehot_n = iota_n == mp[:, None]
        loc = loc_ref[...]
        nn_idx = jnp.sum(jnp.where(onehot_n, loc, 0), axis=1)
        score_s[...] = smax[:, None]
        nnidx_s[...] = nn_idx[:, None]
        rows = emb_ref.shape[0]
        gid = jax.lax.broadcasted_iota(jnp.int32, (_B, 1), 0)[:, 0] * _N + mp
        iota_r = jax.lax.broadcasted_iota(jnp.int32, (_B, rows), 1)
        oh = (iota_r == gid[:, None]).astype(jnp.float32)
        mf_s[...] = jax.lax.dot_general(
            oh, emb_ref[...], (((1,), (0,)), ((), ())),
            preferred_element_type=jnp.float32, precision=_PREC)
        nnt_s[...] = jnp.zeros_like(nnt_s)

    @pl.when(p == 0)
    def _():
        yt = yt_ref[...]
        local = nnidx_s[...][:, 0] - k * _BK
        iota_b = jax.lax.broadcasted_iota(jnp.int32, (_BK, _B), 0)
        oht = (iota_b == local[None, :]).astype(jnp.float32)
        nnt_s[...] = nnt_s[...] + jax.lax.dot_general(
            yt, oht, (((1,), (0,)), ((), ())),
            preferred_element_type=jnp.float32, precision=_PREC)

    @pl.when(p == 1)
    def _():
        @pl.when(k == 0)
        def _():
            topk_s[...] = jnp.full_like(topk_s, jnp.inf)
            topp_s[...] = jnp.zeros_like(topp_s)

        yt = yt_ref[...]
        yn = jnp.sum(yt * yt, axis=0)
        nn = nnt_s[...].T
        mf = mf_s[...]
        xy_nn = jax.lax.dot_general(
            nn, yt, (((1,), (0,)), ((), ())),
            preferred_element_type=jnp.float32, precision=_PREC)
        xy_mf = jax.lax.dot_general(
            mf, yt, (((1,), (0,)), ((), ())),
            preferred_element_type=jnp.float32, precision=_PREC)
        key = yn[None, :] - 2.0 * xy_nn
        mfn = jnp.sum(mf * mf, axis=1, keepdims=True)
        dmf = jnp.clip(mfn + yn[None, :] - 2.0 * xy_mf, 1e-12, None)
        blk_k, blk_p = _topk_merge(key, dmf, kk)
        run_k = topk_s[...][:, :kk]
        run_p = topp_s[...][:, :kk]
        cat_k = jnp.concatenate([run_k, blk_k], axis=1)
        cat_p = jnp.concatenate([run_p, blk_p], axis=1)
        new_k, new_p = _topk_merge(cat_k, cat_p, kk)
        topk_s[...] = jnp.pad(new_k, ((0, 0), (0, topk_s.shape[1] - kk)),
                              constant_values=jnp.inf)
        topp_s[...] = jnp.pad(new_p, ((0, 0), (0, topp_s.shape[1] - kk)))

        @pl.when(k == nk - 1)
        def _():
            d = jnp.sqrt(topp_s[...][:, :kk])
            dmax = jnp.max(d, axis=1, keepdims=True)
            e = jnp.exp(d - dmax)
            p0 = e[:, 0] / jnp.sum(e, axis=1)
            w0 = 1.0 - p0
            pred_ref[...] = (w0 * score_s[...][:, 0])[:, None]


def kernel(embedding, memory_bank):
    rows, d = embedding.shape
    m, _ = memory_bank.shape
    bank_t = memory_bank.T
    grid1 = (rows // _BM, m // _BK)
    scores_col, loc_col = pl.pallas_call(
        _phase1_kernel,
        grid=grid1,
        in_specs=[
            pl.BlockSpec((_BM, d), lambda r, k: (r, 0)),
            pl.BlockSpec((d, _BK), lambda r, k: (0, k)),
        ],
        out_specs=[
            pl.BlockSpec((_BM, 1), lambda r, k: (r, 0)),
            pl.BlockSpec((_BM, 1), lambda r, k: (r, 0)),
        ],
        out_shape=[
            jax.ShapeDtypeStruct((rows, 1), jnp.float32),
            jax.ShapeDtypeStruct((rows, 1), jnp.int32),
        ],
        scratch_shapes=[
            pltpu.VMEM((_BM, 1), jnp.float32),
            pltpu.VMEM((_BM, 1), jnp.int32),
        ],
    )(embedding, bank_t)

    scores = scores_col.reshape(_B, _N)
    locations = loc_col.reshape(_B, _N)

    grid2 = (2, m // _BK)
    pred_col = pl.pallas_call(
        _phase2_kernel,
        grid=grid2,
        in_specs=[
            pl.BlockSpec((_B, _N), lambda p, k: (0, 0)),
            pl.BlockSpec((_B, _N), lambda p, k: (0, 0)),
            pl.BlockSpec((rows, d), lambda p, k: (0, 0)),
            pl.BlockSpec((d, _BK), lambda p, k: (0, k)),
        ],
        out_specs=pl.BlockSpec((_B, 1), lambda p, k: (0, 0)),
        out_shape=jax.ShapeDtypeStruct((_B, 1), jnp.float32),
        scratch_shapes=[
            pltpu.VMEM((_B, 1), jnp.float32),
            pltpu.VMEM((_B, 1), jnp.int32),
            pltpu.VMEM((_B, d), jnp.float32),
            pltpu.VMEM((d, _B), jnp.float32),
            pltpu.VMEM((_B, 16), jnp.float32),
            pltpu.VMEM((_B, 16), jnp.float32),
        ],
    )(scores, locations, embedding, bank_t)

    return scores, pred_col.reshape(_B)


# speedup vs baseline: 2.4638x; 2.4638x over previous
import jax
import jax.numpy as jnp
from jax.experimental import pallas as pl
from jax.experimental.pallas import tpu as pltpu

_B = 8
_N = 512
_NUM_NEIGHBORS = 9

_BM = 512
_BK = 2048

_PREC = jax.lax.Precision.DEFAULT


def _phase1_kernel(x_ref, yt_ref, scores_ref, loc_ref, minv_s, mini_s):
    k = pl.program_id(1)
    nk = pl.num_programs(1)
    x = x_ref[...]
    yt = yt_ref[...]
    yn = jnp.sum(yt * yt, axis=0)
    xy = jax.lax.dot_general(
        x, yt, (((1,), (0,)), ((), ())),
        preferred_element_type=jnp.float32, precision=_PREC)
    part = yn[None, :] - 2.0 * xy
    pmin = jnp.min(part, axis=1)
    iota = jax.lax.broadcasted_iota(jnp.int32, part.shape, 1)
    big = jnp.int32(part.shape[1])
    pidx = jnp.min(jnp.where(part <= pmin[:, None], iota, big), axis=1)
    gidx = k * _BK + pidx

    @pl.when(k == 0)
    def _():
        minv_s[...] = pmin[:, None]
        mini_s[...] = gidx[:, None]

    @pl.when(k > 0)
    def _():
        better = pmin[:, None] < minv_s[...]
        minv_s[...] = jnp.where(better, pmin[:, None], minv_s[...])
        mini_s[...] = jnp.where(better, gidx[:, None], mini_s[...])

    @pl.when(k == nk - 1)
    def _():
        xn = jnp.sum(x * x, axis=1, keepdims=True)
        scores_ref[...] = jnp.sqrt(jnp.clip(xn + minv_s[...], 1e-12, None))
        loc_ref[...] = mini_s[...]


def _topk_merge(keys, pays, kk):
    width = keys.shape[1]
    iota = jax.lax.broadcasted_iota(jnp.int32, keys.shape, 1)
    out_k = []
    out_p = []
    for _ in range(kk):
        m = jnp.min(keys, axis=1)
        sel = jnp.min(jnp.where(keys <= m[:, None], iota, width), axis=1)
        onehot = iota == sel[:, None]
        pay = jnp.sum(jnp.where(onehot, pays, 0.0), axis=1)
        out_k.append(m)
        out_p.append(pay)
        keys = jnp.where(onehot, jnp.float32(jnp.inf), keys)
    return jnp.stack(out_k, axis=1), jnp.stack(out_p, axis=1)


def _phase2_kernel(sc_ref, loc_ref, emb_ref, yt_ref, pred_ref,
                   score_s, nnidx_s, mf_s, nnt_s, topk_s, topp_s):
    p = pl.program_id(0)
    k = pl.program_id(1)
    nk = pl.num_programs(1)
    kk = _NUM_NEIGHBORS

    @pl.when(jnp.logical_and(p == 0, k == 0))
    def _():
        s = sc_ref[...]
        iota_n = jax.lax.broadcasted_iota(jnp.int32, s.shape, 1)
        smax = jnp.max(s, axis=1)
        mp = jnp.min(jnp.where(s >= smax[:, None], iota_n, _N), axis=1)
        onehot_n = iota_n == mp[:, None]
        loc = loc_ref[...]
        nn_idx = jnp.sum(jnp.where(onehot_n, loc, 0), axis=1)
        score_s[...] = smax[:, None]
        nnidx_s[...] = nn_idx[:, None]
        rows = emb_ref.shape[0]
        gid = jax.lax.broadcasted_iota(jnp.int32, (_B, 1), 0)[:, 0] * _N + mp
        iota_r = jax.lax.broadcasted_iota(jnp.int32, (_B, rows), 1)
        oh = (iota_r == gid[:, None]).astype(jnp.float32)
        mf_s[...] = jax.lax.dot_general(
            oh, emb_ref[...], (((1,), (0,)), ((), ())),
            preferred_element_type=jnp.float32, precision=_PREC)
        nnt_s[...] = jnp.zeros_like(nnt_s)

    @pl.when(p == 0)
    def _():
        yt = yt_ref[...]
        local = nnidx_s[...][:, 0] - k * _BK
        iota_b = jax.lax.broadcasted_iota(jnp.int32, (_BK, _B), 0)
        oht = (iota_b == local[None, :]).astype(jnp.float32)
        nnt_s[...] = nnt_s[...] + jax.lax.dot_general(
            yt, oht, (((1,), (0,)), ((), ())),
            preferred_element_type=jnp.float32, precision=_PREC)

    @pl.when(p == 1)
    def _():
        @pl.when(k == 0)
        def _():
            topk_s[...] = jnp.full_like(topk_s, jnp.inf)
            topp_s[...] = jnp.zeros_like(topp_s)

        yt = yt_ref[...]
        yn = jnp.sum(yt * yt, axis=0)
        nn = nnt_s[...].T
        mf = mf_s[...]
        xy_nn = jax.lax.dot_general(
            nn, yt, (((1,), (0,)), ((), ())),
            preferred_element_type=jnp.float32, precision=_PREC)
        xy_mf = jax.lax.dot_general(
            mf, yt, (((1,), (0,)), ((), ())),
            preferred_element_type=jnp.float32, precision=_PREC)
        key = yn[None, :] - 2.0 * xy_nn
        mfn = jnp.sum(mf * mf, axis=1, keepdims=True)
        dmf = jnp.clip(mfn + yn[None, :] - 2.0 * xy_mf, 1e-12, None)
        blk_k, blk_p = _topk_merge(key, dmf, kk)
        run_k = topk_s[...][:, :kk]
        run_p = topp_s[...][:, :kk]
        cat_k = jnp.concatenate([run_k, blk_k], axis=1)
        cat_p = jnp.concatenate([run_p, blk_p], axis=1)
        new_k, new_p = _topk_merge(cat_k, cat_p, kk)
        topk_s[...] = jnp.pad(new_k, ((0, 0), (0, topk_s.shape[1] - kk)),
                              constant_values=jnp.inf)
        topp_s[...] = jnp.pad(new_p, ((0, 0), (0, topp_s.shape[1] - kk)))

        @pl.when(k == nk - 1)
        def _():
            d = jnp.sqrt(topp_s[...][:, :kk])
            dmax = jnp.max(d, axis=1, keepdims=True)
            e = jnp.exp(d - dmax)
            p0 = e[:, 0] / jnp.sum(e, axis=1)
            w0 = 1.0 - p0
            pred_ref[...] = (w0 * score_s[...][:, 0])[:, None]


def kernel(embedding, memory_bank):
    rows, d = embedding.shape
    m, _ = memory_bank.shape
    bank_t = memory_bank.T
    grid1 = (rows // _BM, m // _BK)
    scores_col, loc_col = pl.pallas_call(
        _phase1_kernel,
        grid=grid1,
        in_specs=[
            pl.BlockSpec((_BM, d), lambda r, k: (r, 0)),
            pl.BlockSpec((d, _BK), lambda r, k: (0, k)),
        ],
        out_specs=[
            pl.BlockSpec((_BM, 1), lambda r, k: (r, 0)),
            pl.BlockSpec((_BM, 1), lambda r, k: (r, 0)),
        ],
        out_shape=[
            jax.ShapeDtypeStruct((rows, 1), jnp.float32),
            jax.ShapeDtypeStruct((rows, 1), jnp.int32),
        ],
        scratch_shapes=[
            pltpu.VMEM((_BM, 1), jnp.float32),
            pltpu.VMEM((_BM, 1), jnp.int32),
        ],
    )(embedding, bank_t)

    scores = scores_col.reshape(_B, _N)
    locations = loc_col.reshape(_B, _N)

    grid2 = (2, m // _BK)
    pred_col = pl.pallas_call(
        _phase2_kernel,
        grid=grid2,
        in_specs=[
            pl.BlockSpec((_B, _N), lambda p, k: (0, 0)),
            pl.BlockSpec((_B, _N), lambda p, k: (0, 0)),
            pl.BlockSpec((rows, d), lambda p, k: (0, 0)),
            pl.BlockSpec((d, _BK), lambda p, k: (0, k)),
        ],
        out_specs=pl.BlockSpec((_B, 1), lambda p, k: (0, 0)),
        out_shape=jax.ShapeDtypeStruct((_B, 1), jnp.float32),
        scratch_shapes=[
            pltpu.VMEM((_B, 1), jnp.float32),
            pltpu.VMEM((_B, 1), jnp.int32),
            pltpu.VMEM((_B, d), jnp.float32),
            pltpu.VMEM((d, _B), jnp.float32),
            pltpu.VMEM((_B, 16), jnp.float32),
            pltpu.VMEM((_B, 16), jnp.float32),
        ],
    )(scores, locations, embedding, bank_t)

    return scores, pred_col.reshape(_B)


# speedup vs baseline: 3.1170x; 1.2651x over previous
import jax
import jax.numpy as jnp
from jax.experimental import pallas as pl
from jax.experimental.pallas import tpu as pltpu

_B = 8
_N = 512
_NUM_NEIGHBORS = 9

_BM = 512
_BK = 2048

_PREC = jax.lax.Precision.DEFAULT


def _phase1_kernel(y_ref, xt_ref, scores_ref, minv_s, yn_s):
    k = pl.program_id(0)
    r = pl.program_id(1)
    nk = pl.num_programs(0)
    y = y_ref[...]
    xt = xt_ref[...]

    @pl.when(r == 0)
    def _():
        yn_s[...] = jnp.sum(y * y, axis=1, keepdims=True)

    xyt = jax.lax.dot_general(
        y, xt, (((1,), (0,)), ((), ())),
        preferred_element_type=jnp.float32, precision=_PREC)
    pmin = jnp.min(yn_s[...] - 2.0 * xyt, axis=0)

    @pl.when(k == 0)
    def _():
        minv_s[pl.ds(r, 1), :] = pmin[None, :]

    @pl.when(k > 0)
    def _():
        minv_s[pl.ds(r, 1), :] = jnp.minimum(pmin[None, :],
                                             minv_s[pl.ds(r, 1), :])

    @pl.when(k == nk - 1)
    def _():
        xn = jnp.sum(xt * xt, axis=0)
        best = minv_s[pl.ds(r, 1), :]
        scores_ref[...] = jnp.sqrt(
            jnp.clip(xn[None, :] + best, 1e-12, None))[None]


def _phase2_kernel(sc_ref, emb_ref, yt_ref, pred_ref,
                   score_s, nnidx_s, mf_s, nnt_s, nn_s, key_s, dmf_s):
    p = pl.program_id(0)
    k = pl.program_id(1)
    nk = pl.num_programs(1)
    kk = _NUM_NEIGHBORS
    m = key_s.shape[1]

    @pl.when(jnp.logical_and(p == 0, k == 0))
    def _():
        s = sc_ref[...]
        iota_n = jax.lax.broadcasted_iota(jnp.int32, s.shape, 1)
        smax = jnp.max(s, axis=1)
        mp = jnp.min(jnp.where(s >= smax[:, None], iota_n, _N), axis=1)
        score_s[...] = smax[:, None]
        rows = emb_ref.shape[0]
        gid = jax.lax.broadcasted_iota(jnp.int32, (_B, 1), 0)[:, 0] * _N + mp
        iota_r = jax.lax.broadcasted_iota(jnp.int32, (_B, rows), 1)
        oh = (iota_r == gid[:, None]).astype(jnp.float32)
        mf_s[...] = jax.lax.dot_general(
            oh, emb_ref[...], (((1,), (0,)), ((), ())),
            preferred_element_type=jnp.float32, precision=_PREC)

    @pl.when(p == 0)
    def _():
        yt = yt_ref[...]
        yn = jnp.sum(yt * yt, axis=0)
        mf = mf_s[...]
        mfn = jnp.sum(mf * mf, axis=1, keepdims=True)
        xy = jax.lax.dot_general(
            mf, yt, (((1,), (0,)), ((), ())),
            preferred_element_type=jnp.float32, precision=_PREC)
        dmf_s[:, pl.ds(k * _BK, _BK)] = jnp.clip(
            mfn + yn[None, :] - 2.0 * xy, 1e-12, None)

        @pl.when(k == nk - 1)
        def _():
            dall = dmf_s[...]
            amin = jnp.min(dall, axis=1)
            iota_m = jax.lax.broadcasted_iota(jnp.int32, dall.shape, 1)
            sel = jnp.min(jnp.where(dall <= amin[:, None], iota_m, m), axis=1)
            nnidx_s[...] = sel[:, None]

    @pl.when(p == 1)
    def _():
        yt = yt_ref[...]
        local = nnidx_s[...][:, 0] - k * _BK
        iota_b = jax.lax.broadcasted_iota(jnp.int32, (_BK, _B), 0)
        oht = (iota_b == local[None, :]).astype(jnp.float32)
        acc = jax.lax.dot_general(
            yt, oht, (((1,), (0,)), ((), ())),
            preferred_element_type=jnp.float32, precision=_PREC)

        @pl.when(k == 0)
        def _():
            nnt_s[...] = acc

        @pl.when(k > 0)
        def _():
            nnt_s[...] = nnt_s[...] + acc

        @pl.when(k == nk - 1)
        def _():
            nn_s[...] = nnt_s[...].T

    @pl.when(p == 2)
    def _():
        yt = yt_ref[...]
        yn = jnp.sum(yt * yt, axis=0)
        nn = nn_s[...]
        xy = jax.lax.dot_general(
            nn, yt, (((1,), (0,)), ((), ())),
            preferred_element_type=jnp.float32, precision=_PREC)
        key_s[:, pl.ds(k * _BK, _BK)] = yn[None, :] - 2.0 * xy

        @pl.when(k == nk - 1)
        def _():
            keys = key_s[...]
            pays = dmf_s[...]
            iota_m = jax.lax.broadcasted_iota(jnp.int32, keys.shape, 1)
            ds = []
            for _ in range(kk):
                mn = jnp.min(keys, axis=1)
                sel = jnp.min(jnp.where(keys <= mn[:, None], iota_m, m), axis=1)
                onehot = iota_m == sel[:, None]
                ds.append(jnp.sum(jnp.where(onehot, pays, 0.0), axis=1))
                keys = jnp.where(onehot, jnp.float32(jnp.inf), keys)
            d = jnp.sqrt(jnp.stack(ds, axis=1))
            dmax = jnp.max(d, axis=1, keepdims=True)
            e = jnp.exp(d - dmax)
            p0 = e[:, 0] / jnp.sum(e, axis=1)
            pred_ref[...] = ((1.0 - p0) * score_s[...][:, 0])[:, None]


def kernel(embedding, memory_bank):
    rows, d = embedding.shape
    m, _ = memory_bank.shape
    bank_t = memory_bank.T
    emb_t = embedding.T
    nr = rows // _BM
    grid1 = (m // _BK, nr)
    scores_blk = pl.pallas_call(
        _phase1_kernel,
        grid=grid1,
        in_specs=[
            pl.BlockSpec((_BK, d), lambda k, r: (k, 0)),
            pl.BlockSpec((d, _BM), lambda k, r: (0, r)),
        ],
        out_specs=pl.BlockSpec((1, 1, _BM), lambda k, r: (r, 0, 0)),
        out_shape=jax.ShapeDtypeStruct((nr, 1, _BM), jnp.float32),
        scratch_shapes=[
            pltpu.VMEM((nr, _BM), jnp.float32),
            pltpu.VMEM((_BK, 1), jnp.float32),
        ],
    )(memory_bank, emb_t)

    scores = scores_blk.reshape(_B, _N)

    grid2 = (3, m // _BK)
    pred_col = pl.pallas_call(
        _phase2_kernel,
        grid=grid2,
        in_specs=[
            pl.BlockSpec((_B, _N), lambda p, k: (0, 0)),
            pl.BlockSpec((rows, d), lambda p, k: (0, 0)),
            pl.BlockSpec((d, _BK), lambda p, k: (0, k)),
        ],
        out_specs=pl.BlockSpec((_B, 1), lambda p, k: (0, 0)),
        out_shape=jax.ShapeDtypeStruct((_B, 1), jnp.float32),
        scratch_shapes=[
            pltpu.VMEM((_B, 1), jnp.float32),
            pltpu.VMEM((_B, 1), jnp.int32),
            pltpu.VMEM((_B, d), jnp.float32),
            pltpu.VMEM((d, _B), jnp.float32),
            pltpu.VMEM((_B, d), jnp.float32),
            pltpu.VMEM((_B, m), jnp.float32),
            pltpu.VMEM((_B, m), jnp.float32),
        ],
    )(scores, embedding, bank_t)

    return scores, pred_col.reshape(_B)
